# Initial kernel scaffold; baseline (speedup 1.0000x reference)
import functools
import jax
import jax.numpy as jnp
from jax import lax
from jax.experimental import pallas as pl
from jax.experimental.pallas import tpu as pltpu
from jax.experimental.pallas import tpu_sc as plsc

N_NODES = 50000
N_EDGES = 800000
D = 64
HALF = 32
N_SEEDS = 1024
NS = 16
L = 16

K = 400
EDGES_PER_SUB = N_EDGES // NS
N_BATCH = EDGES_PER_SUB // K

AGG_ROWS = 50176
AGG_PER_SUB = AGG_ROWS // NS
DEG_ROWS = 25088
DEG_PER_SUB = DEG_ROWS // NS
ZCHUNK = 196
HALF_NODES = N_NODES // 2
TRASH = HALF_NODES
OUT_PER_SUB = N_NODES // NS

_mesh = plsc.VectorSubcoreMesh(core_axis_name="c", subcore_axis_name="s")


def _zero_zbuf(zref, ncols):
    z = jnp.zeros((L,), jnp.float32)

    @pl.loop(0, ZCHUNK)
    def _(i):
        for j in range(0, ncols, L):
            zref[i, pl.ds(j, L)] = z


def _edge_pass_body(xref, src_hbm, dst_hbm, agg_out, agg_sh, src_v, dst_v,
                    rows_v, zb32, sid, with_deg, deg_state=None):
    @pl.loop(0, AGG_PER_SUB // ZCHUNK)
    def _(t):
        pltpu.sync_copy(zb32, agg_sh.at[pl.ds(sid * AGG_PER_SUB + t * ZCHUNK,
                                              ZCHUNK)])
    if with_deg:
        deg_sh, deg_out, slot_v, ones_b, zb16, base_node = deg_state

        @pl.loop(0, DEG_PER_SUB // ZCHUNK)
        def _(t):
            pltpu.sync_copy(zb16, deg_sh.at[pl.ds(sid * DEG_PER_SUB +
                                                  t * ZCHUNK, ZCHUNK)])
    plsc.subcore_barrier()

    @pl.loop(0, N_BATCH)
    def _(b):
        base = sid * EDGES_PER_SUB + b * K
        pltpu.sync_copy(src_hbm.at[pl.ds(base, K)], src_v)
        pltpu.sync_copy(dst_hbm.at[pl.ds(base, K)], dst_v)
        pltpu.sync_copy(xref.at[src_v], rows_v)
        pltpu.sync_copy(rows_v, agg_sh.at[dst_v], add=True)
        if with_deg:
            @pl.loop(0, K // L)
            def _(j):
                d = dst_v[pl.ds(j * L, L)]
                loc = d - base_node
                m = (loc >= 0) & (loc < HALF_NODES)
                slot_v[pl.ds(j * L, L)] = jnp.where(m, loc, TRASH)

            pltpu.sync_copy(ones_b, deg_sh.at[slot_v], add=True)

    plsc.subcore_barrier()
    pltpu.sync_copy(agg_sh.at[pl.ds(sid * OUT_PER_SUB, OUT_PER_SUB)],
                    agg_out.at[pl.ds(sid * OUT_PER_SUB, OUT_PER_SUB)])
    if with_deg:
        deg_sh, deg_out, slot_v, ones_b, zb16, base_node = deg_state

        @pl.when(sid < 8)
        def _():
            pltpu.sync_copy(
                deg_sh.at[pl.ds(sid * OUT_PER_SUB * 2, OUT_PER_SUB * 2)],
                deg_out.at[pl.ds(base_node + sid * OUT_PER_SUB * 2,
                                 OUT_PER_SUB * 2)])


def _make_edge_pass(with_deg):
    out_type = [
        jax.ShapeDtypeStruct((N_NODES, HALF), jnp.float32),
        jax.ShapeDtypeStruct((N_NODES, HALF), jnp.float32),
    ]
    scratch = [
        pltpu.VMEM((K,), jnp.int32),
        pltpu.VMEM((K,), jnp.int32),
        pltpu.VMEM((K, HALF), jnp.float32),
        pltpu.VMEM((ZCHUNK, HALF), jnp.float32),
        pltpu.VMEM_SHARED((AGG_ROWS, HALF), jnp.float32),
    ]
    if with_deg:
        out_type.append(jax.ShapeDtypeStruct((N_NODES, L), jnp.float32))
        scratch += [
            pltpu.VMEM((K,), jnp.int32),
            pltpu.VMEM((K, L), jnp.float32),
            pltpu.VMEM((ZCHUNK, L), jnp.float32),
            pltpu.VMEM_SHARED((DEG_ROWS, L), jnp.float32),
        ]

    @functools.partial(pl.kernel, out_type=out_type, mesh=_mesh,
                       scratch_types=scratch)
    def edge_pass(*refs):
        if with_deg:
            (xa, xb, src_hbm, dst_hbm, agg0, agg1, deg_out,
             src_v, dst_v, rows_v, zb32, agg_sh,
             slot_v, ones_b, zb16, deg_sh) = refs
        else:
            (xa, xb, src_hbm, dst_hbm, agg0, agg1,
             src_v, dst_v, rows_v, zb32, agg_sh) = refs
        c = lax.axis_index("c")
        sid = lax.axis_index("s")
        _zero_zbuf(zb32, HALF)
        if with_deg:
            _zero_zbuf(zb16, L)
            one = jnp.full((L,), 1.0, jnp.float32)

            @pl.loop(0, K)
            def _(i):
                ones_b[i, pl.ds(0, L)] = one

        @pl.when(c == 0)
        def _():
            st = (deg_sh, deg_out, slot_v, ones_b, zb16,
                  0) if with_deg else None
            _edge_pass_body(xa, src_hbm, dst_hbm, agg0, agg_sh, src_v,
                            dst_v, rows_v, zb32, sid, with_deg, st)

        @pl.when(c == 1)
        def _():
            st = (deg_sh, deg_out, slot_v, ones_b, zb16,
                  HALF_NODES) if with_deg else None
            _edge_pass_body(xb, src_hbm, dst_hbm, agg1, agg_sh, src_v,
                            dst_v, rows_v, zb32, sid, with_deg, st)

    return edge_pass


_edge_pass_l1 = _make_edge_pass(with_deg=True)
_edge_pass_l2 = _make_edge_pass(with_deg=False)


GATHER_PER_SUB = N_SEEDS // NS


@functools.partial(
    pl.kernel,
    out_type=[
        jax.ShapeDtypeStruct((N_SEEDS, HALF), jnp.float32),
        jax.ShapeDtypeStruct((N_SEEDS, HALF), jnp.float32),
        jax.ShapeDtypeStruct((N_SEEDS, L), jnp.float32),
    ],
    mesh=_mesh,
    scratch_types=[
        pltpu.VMEM((GATHER_PER_SUB,), jnp.int32),
        pltpu.VMEM((GATHER_PER_SUB, HALF), jnp.float32),
        pltpu.VMEM((GATHER_PER_SUB, L), jnp.float32),
    ],
)
def _seed_gather(agg0, agg1, deg, nodes_hbm, ga0, ga1, gdeg,
                 idx_v, rows_v, deg_v):
    c = lax.axis_index("c")
    sid = lax.axis_index("s")
    base = sid * GATHER_PER_SUB
    pltpu.sync_copy(nodes_hbm.at[pl.ds(base, GATHER_PER_SUB)], idx_v)

    @pl.when(c == 0)
    def _():
        pltpu.sync_copy(agg0.at[idx_v], rows_v)
        pltpu.sync_copy(rows_v, ga0.at[pl.ds(base, GATHER_PER_SUB)])
        pltpu.sync_copy(deg.at[idx_v], deg_v)
        pltpu.sync_copy(deg_v, gdeg.at[pl.ds(base, GATHER_PER_SUB)])

    @pl.when(c == 1)
    def _():
        pltpu.sync_copy(agg1.at[idx_v], rows_v)
        pltpu.sync_copy(rows_v, ga1.at[pl.ds(base, GATHER_PER_SUB)])


ROWS_BLK = 200
N_BLKS = N_NODES // ROWS_BLK


def _dense_body(a0_ref, a1_ref, deg_ref, w_ref, b_ref, oa_ref, ob_ref):
    a = jnp.concatenate([a0_ref[...], a1_ref[...]], axis=1)
    dg = jnp.maximum(deg_ref[...][:, 0:1], 1.0)
    h = lax.dot_general(a / dg, w_ref[...],
                        dimension_numbers=(((1,), (1,)), ((), ())),
                        preferred_element_type=jnp.float32)
    h = jnp.maximum(h + b_ref[...], 0.0)
    oa_ref[...] = h[:, :HALF]
    ob_ref[...] = h[:, HALF:]


def _dense_layer1(agg0, agg1, deg, W, b):
    return pl.pallas_call(
        _dense_body,
        grid=(N_BLKS,),
        in_specs=[
            pl.BlockSpec((ROWS_BLK, HALF), lambda i: (i, 0)),
            pl.BlockSpec((ROWS_BLK, HALF), lambda i: (i, 0)),
            pl.BlockSpec((ROWS_BLK, L), lambda i: (i, 0)),
            pl.BlockSpec((D, D), lambda i: (0, 0)),
            pl.BlockSpec((1, D), lambda i: (0, 0)),
        ],
        out_specs=[
            pl.BlockSpec((ROWS_BLK, HALF), lambda i: (i, 0)),
            pl.BlockSpec((ROWS_BLK, HALF), lambda i: (i, 0)),
        ],
        out_shape=[
            jax.ShapeDtypeStruct((N_NODES, HALF), jnp.float32),
            jax.ShapeDtypeStruct((N_NODES, HALF), jnp.float32),
        ],
    )(agg0, agg1, deg, W, b)


def _dense_body2(a0_ref, a1_ref, deg_ref, w_ref, b_ref, o_ref):
    a = jnp.concatenate([a0_ref[...], a1_ref[...]], axis=1)
    dg = jnp.maximum(deg_ref[...][:, 0:1], 1.0)
    h = lax.dot_general(a / dg, w_ref[...],
                        dimension_numbers=(((1,), (1,)), ((), ())),
                        preferred_element_type=jnp.float32)
    o_ref[...] = jnp.maximum(h + b_ref[...], 0.0)


def _dense_layer2(ga0, ga1, gdeg, W, b):
    return pl.pallas_call(
        _dense_body2,
        out_shape=jax.ShapeDtypeStruct((N_SEEDS, D), jnp.float32),
    )(ga0, ga1, gdeg, W, b)


def kernel(x, edge_index, nodes, W1, b1, W2, b2):
    src = edge_index[0].astype(jnp.int32)
    dst = edge_index[1].astype(jnp.int32)
    nodes = nodes.astype(jnp.int32)
    xa = x[:, :HALF]
    xb = x[:, HALF:]

    agg0, agg1, deg = _edge_pass_l1(xa, xb, src, dst)
    hA, hB = _dense_layer1(agg0, agg1, deg, W1, b1.reshape(1, D))
    agg2_0, agg2_1 = _edge_pass_l2(hA, hB, src, dst)
    ga0, ga1, gdeg = _seed_gather(agg2_0, agg2_1, deg, nodes)
    return _dense_layer2(ga0, ga1, gdeg, W2, b2.reshape(1, D))


# speedup vs baseline: 5.6086x; 5.6086x over previous
import functools
import jax
import jax.numpy as jnp
from jax import lax
from jax.experimental import pallas as pl
from jax.experimental.pallas import tpu as pltpu
from jax.experimental.pallas import tpu_sc as plsc

N_NODES = 50000
N_EDGES = 800000
D = 64
HALF = 32
N_SEEDS = 1024
NS = 16
L = 16

K = 400
EDGES_PER_SUB = N_EDGES // NS
N_BATCH = EDGES_PER_SUB // K

AGG_ROWS = 50176
AGG_PER_SUB = AGG_ROWS // NS
DEG_ROWS = 25088
DEG_PER_SUB = DEG_ROWS // NS
ZCHUNK = 392
HALF_NODES = N_NODES // 2
TRASH = HALF_NODES
AGG_TAIL = N_NODES - 15 * AGG_PER_SUB
DEG_TAIL = HALF_NODES - 15 * DEG_PER_SUB

K_DEG = 2000
N_BATCH_DEG = EDGES_PER_SUB // K_DEG

_mesh = plsc.VectorSubcoreMesh(core_axis_name="c", subcore_axis_name="s")
_sc_params = pltpu.CompilerParams(use_tc_tiling_on_sc=False)


def _zero_zbuf(zref, nrows, ncols):
    z = jnp.zeros((L,), jnp.float32)

    @pl.loop(0, nrows)
    def _(i):
        for j in range(0, ncols, L):
            zref[i, pl.ds(j, L)] = z


def _edge_pass_body(xref, src_hbm, dst_hbm, agg_out, agg_sh, src_v, dst_v,
                    rows_v, zb32, sid):
    @pl.loop(0, AGG_PER_SUB // ZCHUNK)
    def _(t):
        pltpu.sync_copy(zb32, agg_sh.at[pl.ds(sid * AGG_PER_SUB + t * ZCHUNK,
                                              ZCHUNK)])
    plsc.subcore_barrier()

    @pl.loop(0, N_BATCH)
    def _(b):
        base = sid * EDGES_PER_SUB + b * K
        pltpu.sync_copy(src_hbm.at[pl.ds(base, K)], src_v)
        pltpu.sync_copy(dst_hbm.at[pl.ds(base, K)], dst_v)
        pltpu.sync_copy(xref.at[src_v], rows_v)
        pltpu.sync_copy(rows_v, agg_sh.at[dst_v], add=True)

    plsc.subcore_barrier()

    @pl.when(sid < 15)
    def _():
        pltpu.sync_copy(agg_sh.at[pl.ds(sid * AGG_PER_SUB, AGG_PER_SUB)],
                        agg_out.at[pl.ds(sid * AGG_PER_SUB, AGG_PER_SUB)])

    @pl.when(sid == 15)
    def _():
        pltpu.sync_copy(agg_sh.at[pl.ds(15 * AGG_PER_SUB, AGG_TAIL)],
                        agg_out.at[pl.ds(15 * AGG_PER_SUB, AGG_TAIL)])


@functools.partial(
    pl.kernel,
    out_type=[
        jax.ShapeDtypeStruct((N_NODES, HALF), jnp.float32),
        jax.ShapeDtypeStruct((N_NODES, HALF), jnp.float32),
    ],
    mesh=_mesh,
    scratch_types=[
        pltpu.VMEM((K,), jnp.int32),
        pltpu.VMEM((K,), jnp.int32),
        pltpu.VMEM((K, HALF), jnp.float32),
        pltpu.VMEM((ZCHUNK, HALF), jnp.float32),
        pltpu.VMEM_SHARED((AGG_ROWS, HALF), jnp.float32),
    ],
    compiler_params=_sc_params,
)
def _edge_pass(xa, xb, src_hbm, dst_hbm, agg0, agg1,
               src_v, dst_v, rows_v, zb32, agg_sh):
    c = lax.axis_index("c")
    sid = lax.axis_index("s")
    _zero_zbuf(zb32, ZCHUNK, HALF)

    @pl.when(c == 0)
    def _():
        _edge_pass_body(xa, src_hbm, dst_hbm, agg0, agg_sh, src_v,
                        dst_v, rows_v, zb32, sid)

    @pl.when(c == 1)
    def _():
        _edge_pass_body(xb, src_hbm, dst_hbm, agg1, agg_sh, src_v,
                        dst_v, rows_v, zb32, sid)


@functools.partial(
    pl.kernel,
    out_type=jax.ShapeDtypeStruct((N_NODES, L), jnp.float32),
    mesh=_mesh,
    scratch_types=[
        pltpu.VMEM((K_DEG,), jnp.int32),
        pltpu.VMEM((K_DEG,), jnp.int32),
        pltpu.VMEM((K_DEG, L), jnp.float32),
        pltpu.VMEM((ZCHUNK, L), jnp.float32),
        pltpu.VMEM_SHARED((DEG_ROWS, L), jnp.float32),
    ],
    compiler_params=_sc_params,
)
def _deg_pass(dst_hbm, deg_out, dst_v, slot_v, ones_b, zb16, deg_sh):
    c = lax.axis_index("c")
    sid = lax.axis_index("s")
    base_node = c * HALF_NODES
    _zero_zbuf(zb16, ZCHUNK, L)
    one = jnp.full((L,), 1.0, jnp.float32)

    @pl.loop(0, K_DEG)
    def _(i):
        ones_b[i, pl.ds(0, L)] = one

    @pl.loop(0, DEG_PER_SUB // ZCHUNK)
    def _(t):
        pltpu.sync_copy(zb16, deg_sh.at[pl.ds(sid * DEG_PER_SUB + t * ZCHUNK,
                                              ZCHUNK)])
    plsc.subcore_barrier()

    @pl.loop(0, N_BATCH_DEG)
    def _(b):
        base = sid * EDGES_PER_SUB + b * K_DEG
        pltpu.sync_copy(dst_hbm.at[pl.ds(base, K_DEG)], dst_v)

        @pl.loop(0, K_DEG // L)
        def _(j):
            d = dst_v[pl.ds(j * L, L)]
            loc = d - base_node
            m = (loc >= 0) & (loc < HALF_NODES)
            slot_v[pl.ds(j * L, L)] = jnp.where(m, loc, TRASH)

        pltpu.sync_copy(ones_b, deg_sh.at[slot_v], add=True)

    plsc.subcore_barrier()

    @pl.when(sid < 15)
    def _():
        pltpu.sync_copy(
            deg_sh.at[pl.ds(sid * DEG_PER_SUB, DEG_PER_SUB)],
            deg_out.at[pl.ds(base_node + sid * DEG_PER_SUB, DEG_PER_SUB)])

    @pl.when(sid == 15)
    def _():
        pltpu.sync_copy(
            deg_sh.at[pl.ds(15 * DEG_PER_SUB, DEG_TAIL)],
            deg_out.at[pl.ds(base_node + 15 * DEG_PER_SUB, DEG_TAIL)])


GATHER_PER_SUB = N_SEEDS // NS


@functools.partial(
    pl.kernel,
    out_type=[
        jax.ShapeDtypeStruct((N_SEEDS, HALF), jnp.float32),
        jax.ShapeDtypeStruct((N_SEEDS, HALF), jnp.float32),
        jax.ShapeDtypeStruct((N_SEEDS, L), jnp.float32),
    ],
    mesh=_mesh,
    scratch_types=[
        pltpu.VMEM((GATHER_PER_SUB,), jnp.int32),
        pltpu.VMEM((GATHER_PER_SUB, HALF), jnp.float32),
        pltpu.VMEM((GATHER_PER_SUB, L), jnp.float32),
    ],
    compiler_params=_sc_params,
)
def _seed_gather(agg0, agg1, deg, nodes_hbm, ga0, ga1, gdeg,
                 idx_v, rows_v, deg_v):
    c = lax.axis_index("c")
    sid = lax.axis_index("s")
    base = sid * GATHER_PER_SUB
    pltpu.sync_copy(nodes_hbm.at[pl.ds(base, GATHER_PER_SUB)], idx_v)

    @pl.when(c == 0)
    def _():
        pltpu.sync_copy(agg0.at[idx_v], rows_v)
        pltpu.sync_copy(rows_v, ga0.at[pl.ds(base, GATHER_PER_SUB)])
        pltpu.sync_copy(deg.at[idx_v], deg_v)
        pltpu.sync_copy(deg_v, gdeg.at[pl.ds(base, GATHER_PER_SUB)])

    @pl.when(c == 1)
    def _():
        pltpu.sync_copy(agg1.at[idx_v], rows_v)
        pltpu.sync_copy(rows_v, ga1.at[pl.ds(base, GATHER_PER_SUB)])


ROWS_BLK = 200
N_BLKS = N_NODES // ROWS_BLK


def _dense_body(a0_ref, a1_ref, deg_ref, w_ref, b_ref, oa_ref, ob_ref):
    a = jnp.concatenate([a0_ref[...], a1_ref[...]], axis=1)
    dg = jnp.maximum(deg_ref[...][:, 0:1], 1.0)
    h = lax.dot_general(a / dg, w_ref[...],
                        dimension_numbers=(((1,), (1,)), ((), ())),
                        preferred_element_type=jnp.float32)
    h = jnp.maximum(h + b_ref[...], 0.0)
    oa_ref[...] = h[:, :HALF]
    ob_ref[...] = h[:, HALF:]


def _dense_layer1(agg0, agg1, deg, W, b):
    return pl.pallas_call(
        _dense_body,
        grid=(N_BLKS,),
        in_specs=[
            pl.BlockSpec((ROWS_BLK, HALF), lambda i: (i, 0)),
            pl.BlockSpec((ROWS_BLK, HALF), lambda i: (i, 0)),
            pl.BlockSpec((ROWS_BLK, L), lambda i: (i, 0)),
            pl.BlockSpec((D, D), lambda i: (0, 0)),
            pl.BlockSpec((1, D), lambda i: (0, 0)),
        ],
        out_specs=[
            pl.BlockSpec((ROWS_BLK, HALF), lambda i: (i, 0)),
            pl.BlockSpec((ROWS_BLK, HALF), lambda i: (i, 0)),
        ],
        out_shape=[
            jax.ShapeDtypeStruct((N_NODES, HALF), jnp.float32),
            jax.ShapeDtypeStruct((N_NODES, HALF), jnp.float32),
        ],
    )(agg0, agg1, deg, W, b)


def _dense_body2(a0_ref, a1_ref, deg_ref, w_ref, b_ref, o_ref):
    a = jnp.concatenate([a0_ref[...], a1_ref[...]], axis=1)
    dg = jnp.maximum(deg_ref[...][:, 0:1], 1.0)
    h = lax.dot_general(a / dg, w_ref[...],
                        dimension_numbers=(((1,), (1,)), ((), ())),
                        preferred_element_type=jnp.float32)
    o_ref[...] = jnp.maximum(h + b_ref[...], 0.0)


def _dense_layer2(ga0, ga1, gdeg, W, b):
    return pl.pallas_call(
        _dense_body2,
        out_shape=jax.ShapeDtypeStruct((N_SEEDS, D), jnp.float32),
    )(ga0, ga1, gdeg, W, b)


def kernel(x, edge_index, nodes, W1, b1, W2, b2):
    src = edge_index[0].astype(jnp.int32)
    dst = edge_index[1].astype(jnp.int32)
    nodes = nodes.astype(jnp.int32)
    xa = x[:, :HALF]
    xb = x[:, HALF:]

    deg = _deg_pass(dst)
    agg0, agg1 = _edge_pass(xa, xb, src, dst)
    hA, hB = _dense_layer1(agg0, agg1, deg, W1, b1.reshape(1, D))
    agg2_0, agg2_1 = _edge_pass(hA, hB, src, dst)
    ga0, ga1, gdeg = _seed_gather(agg2_0, agg2_1, deg, nodes)
    return _dense_layer2(ga0, ga1, gdeg, W2, b2.reshape(1, D))


# speedup vs baseline: 7.5273x; 1.3421x over previous
import functools
import jax
import jax.numpy as jnp
from jax import lax
from jax.experimental import pallas as pl
from jax.experimental.pallas import tpu as pltpu
from jax.experimental.pallas import tpu_sc as plsc

N_NODES = 50000
N_EDGES = 800000
D = 64
HALF = 32
N_SEEDS = 1024
NS = 16
L = 16

K = 400
EDGES_PER_SUB = N_EDGES // NS
N_BATCH = EDGES_PER_SUB // K

AGG_ROWS = 50176
AGG_PER_SUB = AGG_ROWS // NS
DEG_ROWS = 25088
DEG_PER_SUB = DEG_ROWS // NS
ZCHUNK = 392
HALF_NODES = N_NODES // 2
TRASH = HALF_NODES
AGG_TAIL = N_NODES - 15 * AGG_PER_SUB
DEG_TAIL = HALF_NODES - 15 * DEG_PER_SUB

K_DEG = 1000
EDGES_PER_SUB_DEG = N_EDGES // 2 // NS
N_BATCH_DEG = EDGES_PER_SUB_DEG // K_DEG
DEG_FULL_ROWS = AGG_ROWS
DEG_FULL_PER_SUB = DEG_FULL_ROWS // NS
DEG_FULL_TAIL = N_NODES - 15 * DEG_FULL_PER_SUB

_mesh = plsc.VectorSubcoreMesh(core_axis_name="c", subcore_axis_name="s")
_sc_params = pltpu.CompilerParams(use_tc_tiling_on_sc=False)


def _zero_zbuf(zref, nrows, ncols):
    z = jnp.zeros((L,), jnp.float32)

    @pl.loop(0, nrows)
    def _(i):
        for j in range(0, ncols, L):
            zref[i, pl.ds(j, L)] = z


def _edge_pass_body(xref, src_hbm, dst_hbm, agg_out, agg_sh, src_v, dst_v,
                    rows_v, zb32, sid):
    @pl.loop(0, AGG_PER_SUB // ZCHUNK)
    def _(t):
        pltpu.sync_copy(zb32, agg_sh.at[pl.ds(sid * AGG_PER_SUB + t * ZCHUNK,
                                              ZCHUNK)])
    plsc.subcore_barrier()

    @pl.loop(0, N_BATCH)
    def _(b):
        base = sid * EDGES_PER_SUB + b * K
        pltpu.sync_copy(src_hbm.at[pl.ds(base, K)], src_v)
        pltpu.sync_copy(dst_hbm.at[pl.ds(base, K)], dst_v)
        pltpu.sync_copy(xref.at[src_v], rows_v)
        pltpu.sync_copy(rows_v, agg_sh.at[dst_v], add=True)

    plsc.subcore_barrier()

    @pl.when(sid < 15)
    def _():
        pltpu.sync_copy(agg_sh.at[pl.ds(sid * AGG_PER_SUB, AGG_PER_SUB)],
                        agg_out.at[pl.ds(sid * AGG_PER_SUB, AGG_PER_SUB)])

    @pl.when(sid == 15)
    def _():
        pltpu.sync_copy(agg_sh.at[pl.ds(15 * AGG_PER_SUB, AGG_TAIL)],
                        agg_out.at[pl.ds(15 * AGG_PER_SUB, AGG_TAIL)])


@functools.partial(
    pl.kernel,
    out_type=[
        jax.ShapeDtypeStruct((N_NODES, HALF), jnp.float32),
        jax.ShapeDtypeStruct((N_NODES, HALF), jnp.float32),
    ],
    mesh=_mesh,
    scratch_types=[
        pltpu.VMEM((K,), jnp.int32),
        pltpu.VMEM((K,), jnp.int32),
        pltpu.VMEM((K, HALF), jnp.float32),
        pltpu.VMEM((ZCHUNK, HALF), jnp.float32),
        pltpu.VMEM_SHARED((AGG_ROWS, HALF), jnp.float32),
    ],
    compiler_params=_sc_params,
)
def _edge_pass(xa, xb, src_hbm, dst_hbm, agg0, agg1,
               src_v, dst_v, rows_v, zb32, agg_sh):
    c = lax.axis_index("c")
    sid = lax.axis_index("s")
    _zero_zbuf(zb32, ZCHUNK, HALF)

    @pl.when(c == 0)
    def _():
        _edge_pass_body(xa, src_hbm, dst_hbm, agg0, agg_sh, src_v,
                        dst_v, rows_v, zb32, sid)

    @pl.when(c == 1)
    def _():
        _edge_pass_body(xb, src_hbm, dst_hbm, agg1, agg_sh, src_v,
                        dst_v, rows_v, zb32, sid)


@functools.partial(
    pl.kernel,
    out_type=[
        jax.ShapeDtypeStruct((N_NODES, L), jnp.float32),
        jax.ShapeDtypeStruct((N_NODES, L), jnp.float32),
    ],
    mesh=_mesh,
    scratch_types=[
        pltpu.VMEM((K_DEG,), jnp.int32),
        pltpu.VMEM((K_DEG, L), jnp.float32),
        pltpu.VMEM((ZCHUNK, L), jnp.float32),
        pltpu.VMEM_SHARED((DEG_FULL_ROWS, L), jnp.float32),
    ],
    compiler_params=_sc_params,
)
def _deg_pass(dst_hbm, deg_out0, deg_out1, dst_v, ones_b, zb16, deg_sh):
    c = lax.axis_index("c")
    sid = lax.axis_index("s")
    _zero_zbuf(zb16, ZCHUNK, L)
    one = jnp.full((L,), 1.0, jnp.float32)

    @pl.loop(0, K_DEG)
    def _(i):
        ones_b[i, pl.ds(0, L)] = one

    @pl.loop(0, DEG_FULL_PER_SUB // ZCHUNK)
    def _(t):
        pltpu.sync_copy(zb16, deg_sh.at[pl.ds(sid * DEG_FULL_PER_SUB +
                                              t * ZCHUNK, ZCHUNK)])
    plsc.subcore_barrier()

    @pl.loop(0, N_BATCH_DEG)
    def _(b):
        base = (c * NS + sid) * EDGES_PER_SUB_DEG + b * K_DEG
        pltpu.sync_copy(dst_hbm.at[pl.ds(base, K_DEG)], dst_v)
        pltpu.sync_copy(ones_b, deg_sh.at[dst_v], add=True)

    plsc.subcore_barrier()

    def writeout(deg_out):
        @pl.when(sid < 15)
        def _():
            pltpu.sync_copy(
                deg_sh.at[pl.ds(sid * DEG_FULL_PER_SUB, DEG_FULL_PER_SUB)],
                deg_out.at[pl.ds(sid * DEG_FULL_PER_SUB, DEG_FULL_PER_SUB)])

        @pl.when(sid == 15)
        def _():
            pltpu.sync_copy(
                deg_sh.at[pl.ds(15 * DEG_FULL_PER_SUB, DEG_FULL_TAIL)],
                deg_out.at[pl.ds(15 * DEG_FULL_PER_SUB, DEG_FULL_TAIL)])

    @pl.when(c == 0)
    def _():
        writeout(deg_out0)

    @pl.when(c == 1)
    def _():
        writeout(deg_out1)


GATHER_PER_SUB = N_SEEDS // NS


@functools.partial(
    pl.kernel,
    out_type=[
        jax.ShapeDtypeStruct((N_SEEDS, HALF), jnp.float32),
        jax.ShapeDtypeStruct((N_SEEDS, HALF), jnp.float32),
        jax.ShapeDtypeStruct((N_SEEDS, L), jnp.float32),
        jax.ShapeDtypeStruct((N_SEEDS, L), jnp.float32),
    ],
    mesh=_mesh,
    scratch_types=[
        pltpu.VMEM((GATHER_PER_SUB,), jnp.int32),
        pltpu.VMEM((GATHER_PER_SUB, HALF), jnp.float32),
        pltpu.VMEM((GATHER_PER_SUB, L), jnp.float32),
    ],
    compiler_params=_sc_params,
)
def _seed_gather(agg0, agg1, deg0, deg1, nodes_hbm, ga0, ga1, gdeg0, gdeg1,
                 idx_v, rows_v, deg_v):
    c = lax.axis_index("c")
    sid = lax.axis_index("s")
    base = sid * GATHER_PER_SUB
    pltpu.sync_copy(nodes_hbm.at[pl.ds(base, GATHER_PER_SUB)], idx_v)

    @pl.when(c == 0)
    def _():
        pltpu.sync_copy(agg0.at[idx_v], rows_v)
        pltpu.sync_copy(rows_v, ga0.at[pl.ds(base, GATHER_PER_SUB)])
        pltpu.sync_copy(deg0.at[idx_v], deg_v)
        pltpu.sync_copy(deg_v, gdeg0.at[pl.ds(base, GATHER_PER_SUB)])

    @pl.when(c == 1)
    def _():
        pltpu.sync_copy(agg1.at[idx_v], rows_v)
        pltpu.sync_copy(rows_v, ga1.at[pl.ds(base, GATHER_PER_SUB)])
        pltpu.sync_copy(deg1.at[idx_v], deg_v)
        pltpu.sync_copy(deg_v, gdeg1.at[pl.ds(base, GATHER_PER_SUB)])


ROWS_BLK = 200
N_BLKS = N_NODES // ROWS_BLK


def _dense_body(a0_ref, a1_ref, d0_ref, d1_ref, w_ref, b_ref, oa_ref, ob_ref):
    a = jnp.concatenate([a0_ref[...], a1_ref[...]], axis=1)
    dg = jnp.maximum(d0_ref[...][:, 0:1] + d1_ref[...][:, 0:1], 1.0)
    h = lax.dot_general(a / dg, w_ref[...],
                        dimension_numbers=(((1,), (1,)), ((), ())),
                        preferred_element_type=jnp.float32)
    h = jnp.maximum(h + b_ref[...], 0.0)
    oa_ref[...] = h[:, :HALF]
    ob_ref[...] = h[:, HALF:]


def _dense_layer1(agg0, agg1, deg0, deg1, W, b):
    return pl.pallas_call(
        _dense_body,
        grid=(N_BLKS,),
        in_specs=[
            pl.BlockSpec((ROWS_BLK, HALF), lambda i: (i, 0)),
            pl.BlockSpec((ROWS_BLK, HALF), lambda i: (i, 0)),
            pl.BlockSpec((ROWS_BLK, L), lambda i: (i, 0)),
            pl.BlockSpec((ROWS_BLK, L), lambda i: (i, 0)),
            pl.BlockSpec((D, D), lambda i: (0, 0)),
            pl.BlockSpec((1, D), lambda i: (0, 0)),
        ],
        out_specs=[
            pl.BlockSpec((ROWS_BLK, HALF), lambda i: (i, 0)),
            pl.BlockSpec((ROWS_BLK, HALF), lambda i: (i, 0)),
        ],
        out_shape=[
            jax.ShapeDtypeStruct((N_NODES, HALF), jnp.float32),
            jax.ShapeDtypeStruct((N_NODES, HALF), jnp.float32),
        ],
    )(agg0, agg1, deg0, deg1, W, b)


def _dense_body2(a0_ref, a1_ref, d0_ref, d1_ref, w_ref, b_ref, o_ref):
    a = jnp.concatenate([a0_ref[...], a1_ref[...]], axis=1)
    dg = jnp.maximum(d0_ref[...][:, 0:1] + d1_ref[...][:, 0:1], 1.0)
    h = lax.dot_general(a / dg, w_ref[...],
                        dimension_numbers=(((1,), (1,)), ((), ())),
                        preferred_element_type=jnp.float32)
    o_ref[...] = jnp.maximum(h + b_ref[...], 0.0)


def _dense_layer2(ga0, ga1, gdeg0, gdeg1, W, b):
    return pl.pallas_call(
        _dense_body2,
        out_shape=jax.ShapeDtypeStruct((N_SEEDS, D), jnp.float32),
    )(ga0, ga1, gdeg0, gdeg1, W, b)


def kernel(x, edge_index, nodes, W1, b1, W2, b2):
    src = edge_index[0].astype(jnp.int32)
    dst = edge_index[1].astype(jnp.int32)
    nodes = nodes.astype(jnp.int32)
    xa = x[:, :HALF]
    xb = x[:, HALF:]

    deg0, deg1 = _deg_pass(dst)
    agg0, agg1 = _edge_pass(xa, xb, src, dst)
    hA, hB = _dense_layer1(agg0, agg1, deg0, deg1, W1, b1.reshape(1, D))
    agg2_0, agg2_1 = _edge_pass(hA, hB, src, dst)
    ga0, ga1, gdeg0, gdeg1 = _seed_gather(agg2_0, agg2_1, deg0, deg1, nodes)
    return _dense_layer2(ga0, ga1, gdeg0, gdeg1, W2, b2.reshape(1, D))


# speedup vs baseline: 15.7343x; 2.0903x over previous
import functools
import jax
import jax.numpy as jnp
from jax import lax
from jax.experimental import pallas as pl
from jax.experimental.pallas import tpu as pltpu
from jax.experimental.pallas import tpu_sc as plsc

N_NODES = 50000
N_EDGES = 800000
D = 64
HALF = 32
N_SEEDS = 1024
NS = 16
L = 16

K = 400
EDGES_PER_SUB = N_EDGES // NS
N_BATCH = EDGES_PER_SUB // K

AGG_ROWS = 50176
AGG_PER_SUB = AGG_ROWS // NS
DEG_ROWS = 25088
DEG_PER_SUB = DEG_ROWS // NS
ZCHUNK = 392
HALF_NODES = N_NODES // 2
TRASH = HALF_NODES
AGG_TAIL = N_NODES - 15 * AGG_PER_SUB
DEG_TAIL = HALF_NODES - 15 * DEG_PER_SUB

DEG_FULL_ROWS = AGG_ROWS
DEG_FULL_PER_SUB = DEG_FULL_ROWS // NS
DEG_FULL_TAIL = N_NODES - 15 * DEG_FULL_PER_SUB

NW = 2 * NS
EDGES_W = 25088
E_PAD = NW * EDGES_W
CH = 1568
VECS = CH // L
NCHUNK = EDGES_W // CH
K_DEG = 1000
EDGES_PER_SUB_DEG = N_EDGES // 2 // NS
N_BATCH_DEG = EDGES_PER_SUB_DEG // K_DEG
OUTCAP = 25600
BLK = 400
MAXNB = OUTCAP // BLK
MASKN = 50176
TRASH2 = N_NODES

_mesh = plsc.VectorSubcoreMesh(core_axis_name="c", subcore_axis_name="s")
_sc_params = pltpu.CompilerParams(use_tc_tiling_on_sc=False)
_sc_params_nl = pltpu.CompilerParams(use_tc_tiling_on_sc=False,
                                     needs_layout_passes=False)


def _zero_zbuf(zref, nrows, ncols):
    z = jnp.zeros((L,), jnp.float32)

    @pl.loop(0, nrows)
    def _(i):
        for j in range(0, ncols, L):
            zref[i, pl.ds(j, L)] = z


def _edge_pass_body(xref, src_hbm, dst_hbm, agg_out, agg_sh, src_v, dst_v,
                    rows_v, zb32, sid):
    @pl.loop(0, AGG_PER_SUB // ZCHUNK)
    def _(t):
        pltpu.sync_copy(zb32, agg_sh.at[pl.ds(sid * AGG_PER_SUB + t * ZCHUNK,
                                              ZCHUNK)])
    plsc.subcore_barrier()

    @pl.loop(0, N_BATCH)
    def _(b):
        base = sid * EDGES_PER_SUB + b * K
        pltpu.sync_copy(src_hbm.at[pl.ds(base, K)], src_v)
        pltpu.sync_copy(dst_hbm.at[pl.ds(base, K)], dst_v)
        pltpu.sync_copy(xref.at[src_v], rows_v)
        pltpu.sync_copy(rows_v, agg_sh.at[dst_v], add=True)

    plsc.subcore_barrier()

    @pl.when(sid < 15)
    def _():
        pltpu.sync_copy(agg_sh.at[pl.ds(sid * AGG_PER_SUB, AGG_PER_SUB)],
                        agg_out.at[pl.ds(sid * AGG_PER_SUB, AGG_PER_SUB)])

    @pl.when(sid == 15)
    def _():
        pltpu.sync_copy(agg_sh.at[pl.ds(15 * AGG_PER_SUB, AGG_TAIL)],
                        agg_out.at[pl.ds(15 * AGG_PER_SUB, AGG_TAIL)])


@functools.partial(
    pl.kernel,
    out_type=[
        jax.ShapeDtypeStruct((N_NODES, HALF), jnp.float32),
        jax.ShapeDtypeStruct((N_NODES, HALF), jnp.float32),
    ],
    mesh=_mesh,
    scratch_types=[
        pltpu.VMEM((K,), jnp.int32),
        pltpu.VMEM((K,), jnp.int32),
        pltpu.VMEM((K, HALF), jnp.float32),
        pltpu.VMEM((ZCHUNK, HALF), jnp.float32),
        pltpu.VMEM_SHARED((AGG_ROWS, HALF), jnp.float32),
    ],
    compiler_params=_sc_params,
)
def _edge_pass(xa, xb, src_hbm, dst_hbm, agg0, agg1,
               src_v, dst_v, rows_v, zb32, agg_sh):
    c = lax.axis_index("c")
    sid = lax.axis_index("s")
    _zero_zbuf(zb32, ZCHUNK, HALF)

    @pl.when(c == 0)
    def _():
        _edge_pass_body(xa, src_hbm, dst_hbm, agg0, agg_sh, src_v,
                        dst_v, rows_v, zb32, sid)

    @pl.when(c == 1)
    def _():
        _edge_pass_body(xb, src_hbm, dst_hbm, agg1, agg_sh, src_v,
                        dst_v, rows_v, zb32, sid)


@functools.partial(
    pl.kernel,
    out_type=[
        jax.ShapeDtypeStruct((N_NODES,), jnp.float32),
        jax.ShapeDtypeStruct((N_NODES,), jnp.float32),
        jax.ShapeDtypeStruct((N_SEEDS,), jnp.float32),
        jax.ShapeDtypeStruct((N_SEEDS,), jnp.float32),
    ],
    mesh=_mesh,
    scratch_types=[
        pltpu.VMEM((CH,), jnp.int32),
        pltpu.VMEM((MASKN,), jnp.float32),
        pltpu.VMEM((AGG_PER_SUB,), jnp.float32),
        pltpu.VMEM((AGG_PER_SUB,), jnp.float32),
        pltpu.VMEM((N_SEEDS,), jnp.int32),
        pltpu.VMEM((N_SEEDS,), jnp.float32),
        pltpu.VMEM((N_SEEDS,), jnp.float32),
        pltpu.VMEM_SHARED((NS, MASKN), jnp.float32),
        pltpu.VMEM_SHARED((NS, N_SEEDS), jnp.float32),
    ],
    compiler_params=_sc_params_nl,
)
def _deg_pass(dst_hbm, nodes_hbm, deg_out0, deg_out1, gdeg_out0, gdeg_out1,
              dbuf, hist_v, acc_v, tmp_v, nodes_v, part_v, tmp1k_v,
              hist_sh, seed_sh):
    c = lax.axis_index("c")
    sid = lax.axis_index("s")
    w = c * NS + sid
    zf = jnp.zeros((L,), jnp.float32)
    onef = jnp.full((L,), 1.0, jnp.float32)

    @pl.loop(0, MASKN // L)
    def _(i):
        hist_v[pl.ds(i * L, L)] = zf

    @pl.loop(0, NCHUNK)
    def _(ch):
        base = w * EDGES_W + ch * CH
        pltpu.sync_copy(dst_hbm.at[pl.ds(base, CH)], dbuf)

        @pl.loop(0, VECS)
        def _(v):
            d16 = dbuf[pl.ds(v * L, L)]
            plsc.addupdate_scatter(hist_v, [d16], onef)

    pltpu.sync_copy(hist_v, hist_sh.at[sid])
    plsc.subcore_barrier()

    lo = sid * AGG_PER_SUB
    pltpu.sync_copy(hist_sh.at[0, pl.ds(lo, AGG_PER_SUB)], acc_v)
    for k in range(1, NS):
        pltpu.sync_copy(hist_sh.at[k, pl.ds(lo, AGG_PER_SUB)], tmp_v)

        @pl.loop(0, AGG_PER_SUB // L)
        def _(i):
            acc_v[pl.ds(i * L, L)] = (acc_v[pl.ds(i * L, L)] +
                                      tmp_v[pl.ds(i * L, L)])

    pltpu.sync_copy(nodes_hbm, nodes_v)

    @pl.loop(0, N_SEEDS // L)
    def _(i):
        idx16 = nodes_v[pl.ds(i * L, L)]
        local = idx16 - lo
        m = (local >= 0) & (local < AGG_PER_SUB)
        localc = jnp.where(m, local, 0)
        val = plsc.load_gather(acc_v, [localc]) * jnp.where(m, 1.0, 0.0)
        part_v[pl.ds(i * L, L)] = val

    pltpu.sync_copy(part_v, seed_sh.at[sid])

    def writeout(deg_out):
        @pl.when(sid < 15)
        def _():
            pltpu.sync_copy(acc_v, deg_out.at[pl.ds(lo, AGG_PER_SUB)])

        @pl.when(sid == 15)
        def _():
            pltpu.sync_copy(acc_v.at[pl.ds(0, AGG_TAIL)],
                            deg_out.at[pl.ds(15 * AGG_PER_SUB, AGG_TAIL)])

    plsc.subcore_barrier()

    def seed_writeout(gdeg_out):
        pltpu.sync_copy(seed_sh.at[0], part_v)
        for k in range(1, NS):
            pltpu.sync_copy(seed_sh.at[k], tmp1k_v)

            @pl.loop(0, N_SEEDS // L)
            def _(i):
                part_v[pl.ds(i * L, L)] = (part_v[pl.ds(i * L, L)] +
                                           tmp1k_v[pl.ds(i * L, L)])
        pltpu.sync_copy(part_v, gdeg_out)

    @pl.when(c == 0)
    def _():
        writeout(deg_out0)

        @pl.when(sid == 0)
        def _():
            seed_writeout(gdeg_out0)

    @pl.when(c == 1)
    def _():
        writeout(deg_out1)

        @pl.when(sid == 0)
        def _():
            seed_writeout(gdeg_out1)


@functools.partial(
    pl.kernel,
    out_type=[
        jax.ShapeDtypeStruct((NW, OUTCAP), jnp.int32),
        jax.ShapeDtypeStruct((NW, OUTCAP), jnp.int32),
        jax.ShapeDtypeStruct((NW, L), jnp.int32),
    ],
    mesh=_mesh,
    scratch_types=[
        pltpu.VMEM((N_SEEDS,), jnp.int32),
        pltpu.VMEM((MASKN,), jnp.int32),
        pltpu.VMEM((CH,), jnp.int32),
        pltpu.VMEM((CH,), jnp.int32),
        pltpu.VMEM((OUTCAP,), jnp.int32),
        pltpu.VMEM((OUTCAP,), jnp.int32),
        pltpu.VMEM((L,), jnp.int32),
    ],
    compiler_params=_sc_params_nl,
)
def _compact(src_hbm, dst_hbm, nodes_hbm, srclist, dstlist, counts,
             nodes_v, mask_v, sbuf, dbuf, osrc, odst, cnt_v):
    c = lax.axis_index("c")
    sid = lax.axis_index("s")
    w = c * NS + sid
    zero16 = jnp.zeros((L,), jnp.int32)

    @pl.loop(0, MASKN // L)
    def _(i):
        mask_v[pl.ds(i * L, L)] = zero16

    pltpu.sync_copy(nodes_hbm, nodes_v)
    one16 = jnp.full((L,), 1, jnp.int32)

    @pl.loop(0, N_SEEDS // L)
    def _(i):
        idx16 = nodes_v[pl.ds(i * L, L)]
        plsc.store_scatter(mask_v, [idx16], one16)

    def vec_body(v, fl):
        d16 = dbuf[pl.ds(v * L, L)]
        s16 = sbuf[pl.ds(v * L, L)]
        m = plsc.load_gather(mask_v, [d16]) != 0
        plsc.store_compressed(osrc.at[pl.ds(fl, L)], s16, mask=m)
        plsc.store_compressed(odst.at[pl.ds(fl, L)], d16, mask=m)
        return fl + jnp.max(plsc.all_reduce_population_count(m))

    def chunk_body(ch, fl):
        base = w * EDGES_W + ch * CH
        pltpu.sync_copy(src_hbm.at[pl.ds(base, CH)], sbuf)
        pltpu.sync_copy(dst_hbm.at[pl.ds(base, CH)], dbuf)
        return lax.fori_loop(0, VECS, vec_body, fl)

    fill = lax.fori_loop(0, NCHUNK, chunk_body, jnp.int32(0))

    iota16 = lax.broadcasted_iota(jnp.int32, (L,), 0)

    @pl.loop(0, BLK // L + 1)
    def _(j):
        lane = j * L + iota16
        osrc[pl.ds(fill + j * L, L)] = lane
        odst[pl.ds(fill + j * L, L)] = TRASH2 + lax.rem(lane, 176)

    cnt_v[pl.ds(0, L)] = jnp.full((L,), fill, jnp.int32)
    pltpu.sync_copy(cnt_v, counts.at[w])

    nb = (fill + (BLK - 1)) // BLK

    @pl.loop(0, MAXNB)
    def _(b):
        @pl.when(b < nb)
        def _():
            pltpu.sync_copy(osrc.at[pl.ds(b * BLK, BLK)],
                            srclist.at[w, pl.ds(b * BLK, BLK)])
            pltpu.sync_copy(odst.at[pl.ds(b * BLK, BLK)],
                            dstlist.at[w, pl.ds(b * BLK, BLK)])


@functools.partial(
    pl.kernel,
    out_type=[
        jax.ShapeDtypeStruct((NW, OUTCAP), jnp.int32),
        jax.ShapeDtypeStruct((NW, OUTCAP), jnp.int32),
        jax.ShapeDtypeStruct((NW, L), jnp.int32),
    ],
    mesh=_mesh,
    scratch_types=[
        pltpu.VMEM((MASKN,), jnp.int32),
        pltpu.VMEM((BLK,), jnp.int32),
        pltpu.VMEM((CH,), jnp.int32),
        pltpu.VMEM((CH,), jnp.int32),
        pltpu.VMEM((OUTCAP,), jnp.int32),
        pltpu.VMEM((OUTCAP,), jnp.int32),
        pltpu.VMEM((L,), jnp.int32),
        pltpu.VMEM((AGG_PER_SUB,), jnp.int32),
        pltpu.VMEM((AGG_PER_SUB,), jnp.int32),
        pltpu.VMEM_SHARED((4, MASKN), jnp.int32),
    ],
    compiler_params=_sc_params_nl,
)
def _mark_compact(src_hbm, dst_hbm, srclist2, counts2,
                  srclist, dstlist, counts,
                  mask_v, src_v, sbuf, dbuf, osrc, odst, cnt_v,
                  acc_v, tmp_v, mark_sh):
    c = lax.axis_index("c")
    sid = lax.axis_index("s")
    w = c * NS + sid
    zero16 = jnp.zeros((L,), jnp.int32)
    one16 = jnp.full((L,), 1, jnp.int32)

    @pl.loop(0, MASKN // L)
    def _(i):
        mask_v[pl.ds(i * L, L)] = zero16

    @pl.loop(0, AGG_PER_SUB // L)
    def _(i):
        acc_v[pl.ds(i * L, L)] = zero16

    for k in range(2):
        r = 2 * sid + k
        pltpu.sync_copy(counts2.at[r], cnt_v)
        nb = (jnp.max(cnt_v[pl.ds(0, L)]) + (BLK - 1)) // BLK

        def batch(b, carry):
            pltpu.sync_copy(srclist2.at[r, pl.ds(b * BLK, BLK)], src_v)

            @pl.loop(0, BLK // L)
            def _(i):
                idx16 = src_v[pl.ds(i * L, L)]
                plsc.store_scatter(mask_v, [idx16], one16)

            return carry

        lax.fori_loop(0, nb, batch, 0)

    lo = sid * AGG_PER_SUB
    for t in range(4):
        @pl.when(lax.div(sid, 4) == t)
        def _():
            pltpu.sync_copy(mask_v, mark_sh.at[lax.rem(sid, 4)])
        plsc.subcore_barrier()
        for j in range(4):
            pltpu.sync_copy(mark_sh.at[j, pl.ds(lo, AGG_PER_SUB)], tmp_v)

            @pl.loop(0, AGG_PER_SUB // L)
            def _(i):
                acc_v[pl.ds(i * L, L)] = (acc_v[pl.ds(i * L, L)] +
                                          tmp_v[pl.ds(i * L, L)])
        plsc.subcore_barrier()

    pltpu.sync_copy(acc_v, mark_sh.at[0, pl.ds(lo, AGG_PER_SUB)])
    plsc.subcore_barrier()
    pltpu.sync_copy(mark_sh.at[0], mask_v)

    def vec_body(v, fl):
        d16 = dbuf[pl.ds(v * L, L)]
        s16 = sbuf[pl.ds(v * L, L)]
        m = plsc.load_gather(mask_v, [d16]) != 0
        plsc.store_compressed(osrc.at[pl.ds(fl, L)], s16, mask=m)
        plsc.store_compressed(odst.at[pl.ds(fl, L)], d16, mask=m)
        return fl + jnp.max(plsc.all_reduce_population_count(m))

    def chunk_body(ch, fl):
        base = w * EDGES_W + ch * CH
        pltpu.sync_copy(src_hbm.at[pl.ds(base, CH)], sbuf)
        pltpu.sync_copy(dst_hbm.at[pl.ds(base, CH)], dbuf)
        return lax.fori_loop(0, VECS, vec_body, fl)

    fill = lax.fori_loop(0, NCHUNK, chunk_body, jnp.int32(0))

    iota16 = lax.broadcasted_iota(jnp.int32, (L,), 0)

    @pl.loop(0, BLK // L + 1)
    def _(j):
        lane = j * L + iota16
        osrc[pl.ds(fill + j * L, L)] = lane
        odst[pl.ds(fill + j * L, L)] = TRASH2 + lax.rem(lane, 176)

    cnt_v[pl.ds(0, L)] = jnp.full((L,), fill, jnp.int32)
    pltpu.sync_copy(cnt_v, counts.at[w])

    nb = (fill + (BLK - 1)) // BLK

    @pl.loop(0, MAXNB)
    def _(b):
        @pl.when(b < nb)
        def _():
            pltpu.sync_copy(osrc.at[pl.ds(b * BLK, BLK)],
                            srclist.at[w, pl.ds(b * BLK, BLK)])
            pltpu.sync_copy(odst.at[pl.ds(b * BLK, BLK)],
                            dstlist.at[w, pl.ds(b * BLK, BLK)])


def _masked_l1_body(xref, srclist, dstlist, counts, agg_out, agg_sh,
                    src_v, dst_v, rows_v, zb32, cnt_v, sid):
    @pl.loop(0, AGG_PER_SUB // ZCHUNK)
    def _(t):
        pltpu.sync_copy(zb32, agg_sh.at[pl.ds(sid * AGG_PER_SUB + t * ZCHUNK,
                                              ZCHUNK)])
    plsc.subcore_barrier()

    for k in range(2):
        r = 2 * sid + k
        pltpu.sync_copy(counts.at[r], cnt_v)
        nb = (jnp.max(cnt_v[pl.ds(0, L)]) + (BLK - 1)) // BLK

        def batch(b, carry):
            pltpu.sync_copy(srclist.at[r, pl.ds(b * BLK, BLK)], src_v)
            pltpu.sync_copy(dstlist.at[r, pl.ds(b * BLK, BLK)], dst_v)
            pltpu.sync_copy(xref.at[src_v], rows_v)
            pltpu.sync_copy(rows_v, agg_sh.at[dst_v], add=True)
            return carry

        lax.fori_loop(0, nb, batch, 0)

    plsc.subcore_barrier()

    @pl.when(sid < 15)
    def _():
        pltpu.sync_copy(agg_sh.at[pl.ds(sid * AGG_PER_SUB, AGG_PER_SUB)],
                        agg_out.at[pl.ds(sid * AGG_PER_SUB, AGG_PER_SUB)])

    @pl.when(sid == 15)
    def _():
        pltpu.sync_copy(agg_sh.at[pl.ds(15 * AGG_PER_SUB, AGG_TAIL)],
                        agg_out.at[pl.ds(15 * AGG_PER_SUB, AGG_TAIL)])


@functools.partial(
    pl.kernel,
    out_type=[
        jax.ShapeDtypeStruct((N_NODES, HALF), jnp.float32),
        jax.ShapeDtypeStruct((N_NODES, HALF), jnp.float32),
    ],
    mesh=_mesh,
    scratch_types=[
        pltpu.VMEM((BLK,), jnp.int32),
        pltpu.VMEM((BLK,), jnp.int32),
        pltpu.VMEM((BLK, HALF), jnp.float32),
        pltpu.VMEM((ZCHUNK, HALF), jnp.float32),
        pltpu.VMEM((L,), jnp.int32),
        pltpu.VMEM_SHARED((AGG_ROWS, HALF), jnp.float32),
    ],
    compiler_params=_sc_params_nl,
)
def _masked_l1(xa, xb, srclist, dstlist, counts, agg0, agg1,
               src_v, dst_v, rows_v, zb32, cnt_v, agg_sh):
    c = lax.axis_index("c")
    sid = lax.axis_index("s")
    _zero_zbuf(zb32, ZCHUNK, HALF)

    @pl.when(c == 0)
    def _():
        _masked_l1_body(xa, srclist, dstlist, counts, agg0, agg_sh,
                        src_v, dst_v, rows_v, zb32, cnt_v, sid)

    @pl.when(c == 1)
    def _():
        _masked_l1_body(xb, srclist, dstlist, counts, agg1, agg_sh,
                        src_v, dst_v, rows_v, zb32, cnt_v, sid)


GATHER_PER_SUB = N_SEEDS // NS


def _masked_l2_body(xref, ga, srclist, dstlist, counts, nodes_hbm,
                    agg_sh, idxs_v, zrow_v, grow_v, cnt_v,
                    src_v, dst_v, rows_v, sid):
    base = sid * GATHER_PER_SUB
    pltpu.sync_copy(nodes_hbm.at[pl.ds(base, GATHER_PER_SUB)], idxs_v)
    pltpu.sync_copy(zrow_v, agg_sh.at[idxs_v])

    @pl.when(sid == 0)
    def _():
        pltpu.sync_copy(zrow_v.at[pl.ds(0, 8)],
                        agg_sh.at[pl.ds(TRASH2, 8)])

    plsc.subcore_barrier()

    for k in range(2):
        r = 2 * sid + k
        pltpu.sync_copy(counts.at[r], cnt_v)
        nb = (jnp.max(cnt_v[pl.ds(0, L)]) + (BLK - 1)) // BLK

        def batch(b, carry):
            pltpu.sync_copy(srclist.at[r, pl.ds(b * BLK, BLK)], src_v)
            pltpu.sync_copy(dstlist.at[r, pl.ds(b * BLK, BLK)], dst_v)
            pltpu.sync_copy(xref.at[src_v], rows_v)
            pltpu.sync_copy(rows_v, agg_sh.at[dst_v], add=True)
            return carry

        lax.fori_loop(0, nb, batch, 0)

    plsc.subcore_barrier()
    pltpu.sync_copy(agg_sh.at[idxs_v], grow_v)
    pltpu.sync_copy(grow_v, ga.at[pl.ds(base, GATHER_PER_SUB)])


@functools.partial(
    pl.kernel,
    out_type=[
        jax.ShapeDtypeStruct((N_SEEDS, HALF), jnp.float32),
        jax.ShapeDtypeStruct((N_SEEDS, HALF), jnp.float32),
    ],
    mesh=_mesh,
    scratch_types=[
        pltpu.VMEM((GATHER_PER_SUB,), jnp.int32),
        pltpu.VMEM((GATHER_PER_SUB, HALF), jnp.float32),
        pltpu.VMEM((GATHER_PER_SUB, HALF), jnp.float32),
        pltpu.VMEM((L,), jnp.int32),
        pltpu.VMEM((BLK,), jnp.int32),
        pltpu.VMEM((BLK,), jnp.int32),
        pltpu.VMEM((BLK, HALF), jnp.float32),
        pltpu.VMEM_SHARED((AGG_ROWS, HALF), jnp.float32),
    ],
    compiler_params=_sc_params_nl,
)
def _masked_l2(hA, hB, srclist, dstlist, counts, nodes_hbm,
               ga0, ga1,
               idxs_v, zrow_v, grow_v, cnt_v, src_v, dst_v, rows_v,
               agg_sh):
    c = lax.axis_index("c")
    sid = lax.axis_index("s")
    z = jnp.zeros((L,), jnp.float32)

    @pl.loop(0, GATHER_PER_SUB)
    def _(i):
        zrow_v[i, pl.ds(0, L)] = z
        zrow_v[i, pl.ds(L, L)] = z

    @pl.when(c == 0)
    def _():
        _masked_l2_body(hA, ga0, srclist, dstlist, counts,
                        nodes_hbm, agg_sh, idxs_v, zrow_v, grow_v,
                        cnt_v, src_v, dst_v, rows_v, sid)

    @pl.when(c == 1)
    def _():
        _masked_l2_body(hB, ga1, srclist, dstlist, counts,
                        nodes_hbm, agg_sh, idxs_v, zrow_v, grow_v,
                        cnt_v, src_v, dst_v, rows_v, sid)


ROWS_BLK = 2000
N_BLKS = N_NODES // ROWS_BLK


def _dense_body(a0_ref, a1_ref, d0_ref, d1_ref, w_ref, b_ref, oa_ref, ob_ref):
    a = jnp.concatenate([a0_ref[...], a1_ref[...]], axis=1)
    dg = jnp.maximum(d0_ref[...] + d1_ref[...], 1.0)
    h = lax.dot_general(a / dg, w_ref[...],
                        dimension_numbers=(((1,), (1,)), ((), ())),
                        preferred_element_type=jnp.float32)
    h = jnp.maximum(h + b_ref[...], 0.0)
    oa_ref[...] = h[:, :HALF]
    ob_ref[...] = h[:, HALF:]


def _dense_layer1(agg0, agg1, deg0, deg1, W, b):
    return pl.pallas_call(
        _dense_body,
        grid=(N_BLKS,),
        in_specs=[
            pl.BlockSpec((ROWS_BLK, HALF), lambda i: (i, 0)),
            pl.BlockSpec((ROWS_BLK, HALF), lambda i: (i, 0)),
            pl.BlockSpec((ROWS_BLK, 1), lambda i: (i, 0)),
            pl.BlockSpec((ROWS_BLK, 1), lambda i: (i, 0)),
            pl.BlockSpec((D, D), lambda i: (0, 0)),
            pl.BlockSpec((1, D), lambda i: (0, 0)),
        ],
        out_specs=[
            pl.BlockSpec((ROWS_BLK, HALF), lambda i: (i, 0)),
            pl.BlockSpec((ROWS_BLK, HALF), lambda i: (i, 0)),
        ],
        out_shape=[
            jax.ShapeDtypeStruct((N_NODES, HALF), jnp.float32),
            jax.ShapeDtypeStruct((N_NODES, HALF), jnp.float32),
        ],
    )(agg0, agg1, deg0, deg1, W, b)


def _dense_body2(a0_ref, a1_ref, d0_ref, d1_ref, w_ref, b_ref, o_ref):
    a = jnp.concatenate([a0_ref[...], a1_ref[...]], axis=1)
    dg = jnp.maximum(d0_ref[...] + d1_ref[...], 1.0)
    h = lax.dot_general(a / dg, w_ref[...],
                        dimension_numbers=(((1,), (1,)), ((), ())),
                        preferred_element_type=jnp.float32)
    o_ref[...] = jnp.maximum(h + b_ref[...], 0.0)


def _dense_layer2(ga0, ga1, gdeg0, gdeg1, W, b):
    return pl.pallas_call(
        _dense_body2,
        out_shape=jax.ShapeDtypeStruct((N_SEEDS, D), jnp.float32),
    )(ga0, ga1, gdeg0, gdeg1, W, b)


def kernel(x, edge_index, nodes, W1, b1, W2, b2):
    src = edge_index[0].astype(jnp.int32)
    dst = edge_index[1].astype(jnp.int32)
    nodes = nodes.astype(jnp.int32)
    xa = x[:, :HALF]
    xb = x[:, HALF:]

    pad = E_PAD - N_EDGES
    srcp = jnp.concatenate([src, jnp.zeros((pad,), jnp.int32)])
    dstp = jnp.concatenate([dst, jnp.full((pad,), TRASH2, jnp.int32)])

    deg0, deg1, gd0, gd1 = _deg_pass(dstp, nodes)
    srclist, dstlist, counts = _compact(srcp, dstp, nodes)
    sl1, dl1, cnt1 = _mark_compact(srcp, dstp, srclist, counts)
    agg0, agg1 = _masked_l1(xa, xb, sl1, dl1, cnt1)
    hA, hB = _dense_layer1(agg0, agg1, deg0.reshape(N_NODES, 1),
                           deg1.reshape(N_NODES, 1), W1, b1.reshape(1, D))
    ga0, ga1 = _masked_l2(hA, hB, srclist, dstlist, counts, nodes)
    return _dense_layer2(ga0, ga1, gd0.reshape(N_SEEDS, 1),
                         gd1.reshape(N_SEEDS, 1), W2, b2.reshape(1, D))
